# Initial kernel scaffold; baseline (speedup 1.0000x reference)
import jax
import jax.numpy as jnp
from jax.experimental import pallas as pl


def kernel(token_ids, table):
    raise NotImplementedError("write your pallas kernel here")


# speedup vs baseline: 4.5819x; 4.5819x over previous
import functools

import jax
import jax.numpy as jnp
from jax import lax
from jax.experimental import pallas as pl
from jax.experimental.pallas import tpu as pltpu
from jax.experimental.pallas import tpu_sc as plsc

NC = 2
NS = 16
NW = NC * NS

D = 64
B = 4096 * 50
CH = 128
PER_W = B // NW
NCH = PER_W // CH
NBUF = 5
NGRP = NCH // NBUF


def _emb_body(idx_hbm, table_hbm, out_hbm, idx_v, rows_v, sems):
    wid = lax.axis_index("s") * NC + lax.axis_index("c")
    base = wid * PER_W
    pltpu.sync_copy(idx_hbm.at[wid], idx_v)

    def group(g, _):
        j0 = g * NBUF
        handles = [
            pltpu.async_copy(
                table_hbm.at[idx_v.at[j0 + b]], rows_v.at[b], sems.at[b]
            )
            for b in range(NBUF)
        ]
        for b in range(NBUF):
            handles[b].wait()
            pltpu.sync_copy(
                rows_v.at[b], out_hbm.at[pl.ds(base + (j0 + b) * CH, CH)]
            )
        return 0

    lax.fori_loop(0, NGRP, group, 0)


@jax.jit
def kernel(token_ids, table):
    idx = token_ids.reshape(-1).astype(jnp.int32).reshape(NW, NCH, CH)
    table = table.astype(jnp.float32)
    run = functools.partial(
        pl.kernel,
        out_type=jax.ShapeDtypeStruct((B, D), jnp.float32),
        mesh=plsc.VectorSubcoreMesh(core_axis_name="c", subcore_axis_name="s"),
        scratch_types=[
            pltpu.VMEM((NCH, CH), jnp.int32),
            pltpu.VMEM((NBUF, CH, D), jnp.float32),
            pltpu.SemaphoreType.DMA((NBUF,)),
        ],
        compiler_params=pltpu.CompilerParams(use_tc_tiling_on_sc=False),
    )(_emb_body)
    out = run(idx, table)
    return out.reshape(token_ids.shape[0], token_ids.shape[1], D)
